# baseline (speedup 1.0000x reference)
import functools

import jax
import jax.numpy as jnp
from jax import lax
from jax.experimental import pallas as pl
from jax.experimental.pallas import tpu as pltpu
from jax.experimental.pallas import tpu_sc as plsc

NC = 2
NS = 16
L = 16
NW = NC * NS
CH = 128
G = 8

DIS_W = 16


def _mesh():
    return plsc.VectorSubcoreMesh(core_axis_name="c", subcore_axis_name="s")


_SC_PARAMS = pltpu.CompilerParams(use_tc_tiling_on_sc=False,
                                  needs_layout_passes=False)


def _fill_rows(buf, rows, Dw, value):
    vec = jnp.full((L,), value, jnp.float32)

    @pl.loop(0, rows)
    def _(r):
        @pl.loop(0, Dw, step=L)
        def _(col):
            buf.at[r, pl.ds(col, L)][...] = vec


def _rsqrt_nr(d):
    i = plsc.bitcast(d, jnp.int32)
    i = jnp.full(d.shape, 0x5F3759DF, jnp.int32) - lax.shift_right_logical(i, 1)
    y = plsc.bitcast(i, jnp.float32)
    half_d = 0.5 * d
    for _ in range(3):
        y = y * (1.5 - half_d * y * y)
    return y


def _agg_pipeline(table, src_v, dst_v, acc, rows_a, rows_b,
                  sem_ga, sem_gb, sem_sa, sem_sb, S):
    def fire_g(sc_idx, rows, sem):
        for g in range(G):
            pltpu.async_copy(table.at[src_v.at[sc_idx * G + g]],
                             rows.at[pl.ds(g * CH, CH)], sem)

    def drain_g(rows, sem):
        for g in range(G):
            pltpu.make_async_copy(table.at[src_v.at[0]],
                                  rows.at[pl.ds(g * CH, CH)], sem).wait()

    def fire_s(sc_idx, rows, sem):
        for g in range(G):
            pltpu.async_copy(rows.at[pl.ds(g * CH, CH)],
                             acc.at[dst_v.at[sc_idx * G + g]], sem, add=True)

    def drain_s(rows, sem):
        for g in range(G):
            pltpu.make_async_copy(rows.at[pl.ds(g * CH, CH)],
                                  acc.at[dst_v.at[0]], sem).wait()

    fire_g(0, rows_a, sem_ga)

    @pl.loop(0, S // 2)
    def _(t):
        s0 = t * 2
        s1 = s0 + 1
        drain_g(rows_a, sem_ga)

        @pl.when(t > 0)
        def _():
            drain_s(rows_b, sem_sb)

        fire_g(s1, rows_b, sem_gb)
        fire_s(s0, rows_a, sem_sa)
        drain_g(rows_b, sem_gb)
        drain_s(rows_a, sem_sa)

        @pl.when(t + 1 < S // 2)
        def _():
            fire_g(s0 + 2, rows_a, sem_ga)

        fire_s(s1, rows_b, sem_sb)

    drain_s(rows_b, sem_sb)


def _make_deg_kernel(NP, cpw):
    rps = NP // NS

    @functools.partial(
        pl.kernel,
        out_type=jax.ShapeDtypeStruct((NC, NP, DIS_W), jnp.float32),
        mesh=_mesh(),
        scratch_types=[
            pltpu.VMEM((cpw, CH), jnp.int32),
            pltpu.VMEM((CH, DIS_W), jnp.float32),
            pltpu.VMEM((rps, DIS_W), jnp.float32),
            pltpu.VMEM_SHARED((NP, DIS_W), jnp.float32),
            pltpu.SemaphoreType.DMA,
        ],
        compiler_params=_SC_PARAMS,
    )
    def deg_kernel(dst_hbm, out_hbm, dst_v, ones_v, zbuf, acc, sem):
        c = lax.axis_index("c")
        s = lax.axis_index("s")
        _fill_rows(ones_v, CH, DIS_W, 1.0)
        _fill_rows(zbuf, rps, DIS_W, 0.0)
        row0 = pl.multiple_of(s * rps, 8)
        pltpu.sync_copy(zbuf, acc.at[pl.ds(row0, rps)])
        plsc.subcore_barrier()

        w = c * NS + s
        pltpu.sync_copy(dst_hbm.at[w], dst_v)

        @pl.loop(0, cpw)
        def _(j):
            pltpu.async_copy(ones_v, acc.at[dst_v.at[j]], sem, add=True)

        @pl.loop(0, cpw)
        def _(j):
            pltpu.make_async_copy(ones_v, acc.at[dst_v.at[0]], sem).wait()

        plsc.subcore_barrier()
        pltpu.sync_copy(acc.at[pl.ds(row0, rps)],
                        out_hbm.at[c].at[pl.ds(row0, rps)])

    return deg_kernel


def _make_layer1_kernel(NP, cpw):
    rps = NP // NS
    S = cpw // G

    @functools.partial(
        pl.kernel,
        out_type=(
            jax.ShapeDtypeStruct((NC, NP, L), jnp.float32),
            jax.ShapeDtypeStruct((NC, NP, L), jnp.float32),
            jax.ShapeDtypeStruct((NP, DIS_W), jnp.float32),
        ),
        mesh=_mesh(),
        scratch_types=[
            pltpu.VMEM((cpw, CH), jnp.int32),
            pltpu.VMEM((cpw, CH), jnp.int32),
            pltpu.VMEM((G * CH, L), jnp.float32),
            pltpu.VMEM((G * CH, L), jnp.float32),
            pltpu.VMEM((NP // NS, 2 * L), jnp.float32),
            pltpu.VMEM((NP // NS, DIS_W), jnp.float32),
            pltpu.VMEM((NP // NS, DIS_W), jnp.float32),
            pltpu.VMEM_SHARED((NP, L), jnp.float32),
            pltpu.VMEM_SHARED((NP, L), jnp.float32),
            pltpu.VMEM_SHARED((NP, L), jnp.float32),
            pltpu.SemaphoreType.DMA,
            pltpu.SemaphoreType.DMA,
            pltpu.SemaphoreType.DMA,
            pltpu.SemaphoreType.DMA,
        ],
        compiler_params=_SC_PARAMS,
    )
    def layer1_kernel(h1_hbm, degp_hbm, src_hbm, dst_hbm,
                      outa_hbm, outb_hbm, dis_hbm,
                      src_v, dst_v, rows_a, rows_b, h_v, dis_v, deg1_v,
                      acc, table_a, table_b,
                      sem_ga, sem_gb, sem_sa, sem_sb):
        c = lax.axis_index("c")
        s = lax.axis_index("s")
        row0 = pl.multiple_of(s * rps, 8)
        acc_slice = acc.at[pl.ds(row0, rps)]

        _fill_rows(rows_a, rps, L, 0.0)
        pltpu.sync_copy(rows_a.at[pl.ds(0, rps)], acc_slice)

        pltpu.sync_copy(degp_hbm.at[0].at[pl.ds(row0, rps)], dis_v)
        pltpu.sync_copy(degp_hbm.at[1].at[pl.ds(row0, rps)], deg1_v)
        pltpu.sync_copy(h1_hbm.at[pl.ds(row0, rps)], h_v)

        @pl.loop(0, rps)
        def _(r):
            d = (dis_v.at[r, pl.ds(0, L)][...]
                 + deg1_v.at[r, pl.ds(0, L)][...] + 1.0)
            dis = _rsqrt_nr(d)
            dis_v.at[r, pl.ds(0, L)][...] = dis
            rows_a.at[r, pl.ds(0, L)][...] = h_v.at[r, pl.ds(0, L)][...] * dis
            rows_b.at[r, pl.ds(0, L)][...] = h_v.at[r, pl.ds(L, L)][...] * dis

        pltpu.sync_copy(rows_a.at[pl.ds(0, rps)], table_a.at[pl.ds(row0, rps)])
        pltpu.sync_copy(rows_b.at[pl.ds(0, rps)], table_b.at[pl.ds(row0, rps)])

        @pl.when(c == s % 2)
        def _():
            pltpu.sync_copy(dis_v, dis_hbm.at[pl.ds(row0, rps)])

        w = c * NS + s
        pltpu.sync_copy(src_hbm.at[w], src_v)
        pltpu.sync_copy(dst_hbm.at[w], dst_v)
        plsc.subcore_barrier()

        _agg_pipeline(table_a, src_v, dst_v, acc, rows_a, rows_b,
                      sem_ga, sem_gb, sem_sa, sem_sb, S)

        plsc.subcore_barrier()
        pltpu.sync_copy(acc_slice, outa_hbm.at[c].at[pl.ds(row0, rps)])

        _fill_rows(rows_a, rps, L, 0.0)
        pltpu.sync_copy(rows_a.at[pl.ds(0, rps)], acc_slice)
        plsc.subcore_barrier()

        _agg_pipeline(table_b, src_v, dst_v, acc, rows_a, rows_b,
                      sem_ga, sem_gb, sem_sa, sem_sb, S)

        plsc.subcore_barrier()
        pltpu.sync_copy(acc_slice, outb_hbm.at[c].at[pl.ds(row0, rps)])

    return layer1_kernel


def _make_agg2_final_kernel(NP, cpw, Dw):
    rps = NP // NS
    S = 2 * cpw // G

    @functools.partial(
        pl.kernel,
        out_type=jax.ShapeDtypeStruct((NP, Dw), jnp.float32),
        mesh=_mesh(),
        scratch_types=[
            pltpu.VMEM((2 * cpw, CH), jnp.int32),
            pltpu.VMEM((2 * cpw, CH), jnp.int32),
            pltpu.VMEM((G * CH, Dw), jnp.float32),
            pltpu.VMEM((G * CH, Dw), jnp.float32),
            pltpu.VMEM((NP // NS, DIS_W), jnp.float32),
            pltpu.VMEM((8, Dw), jnp.float32),
            pltpu.VMEM_SHARED((NP, Dw), jnp.float32),
            pltpu.VMEM_SHARED((NP, Dw), jnp.float32),
            pltpu.SemaphoreType.DMA,
            pltpu.SemaphoreType.DMA,
            pltpu.SemaphoreType.DMA,
            pltpu.SemaphoreType.DMA,
        ],
        compiler_params=_SC_PARAMS,
    )
    def agg_kernel(table_hbm, dis_hbm, src_hbm, dst_hbm, b2_hbm, out_hbm,
                   src_v, dst_v, rows_a, rows_b, dis_v, b2_v, acc, table,
                   sem_ga, sem_gb, sem_sa, sem_sb):
        c = lax.axis_index("c")
        s = lax.axis_index("s")
        row0 = pl.multiple_of(s * rps, 8)

        _fill_rows(rows_a, rps, Dw, 0.0)
        pltpu.sync_copy(rows_a.at[pl.ds(0, rps)], acc.at[pl.ds(row0, rps)])
        pltpu.sync_copy(table_hbm.at[pl.ds(row0, rps)],
                        table.at[pl.ds(row0, rps)])

        pltpu.sync_copy(src_hbm.at[s], src_v)
        pltpu.sync_copy(dst_hbm.at[s], dst_v)
        plsc.subcore_barrier()

        _agg_pipeline(table, src_v, dst_v, acc, rows_a, rows_b,
                      sem_ga, sem_gb, sem_sa, sem_sb, S)

        plsc.subcore_barrier()

        @pl.when(c == s % 2)
        def _():
            pltpu.sync_copy(acc.at[pl.ds(row0, rps)], rows_a.at[pl.ds(0, rps)])
            pltpu.sync_copy(table.at[pl.ds(row0, rps)],
                            rows_b.at[pl.ds(0, rps)])
            pltpu.sync_copy(dis_hbm.at[pl.ds(row0, rps)], dis_v)
            pltpu.sync_copy(b2_hbm, b2_v)

            @pl.loop(0, rps)
            def _(r):
                dis = dis_v.at[r, pl.ds(0, L)][...]
                for col in range(0, Dw, L):
                    agg = rows_a.at[r, pl.ds(col, L)][...]
                    hh2 = rows_b.at[r, pl.ds(col, L)][...]
                    b2 = b2_v.at[0, pl.ds(col, L)][...]
                    rows_a.at[r, pl.ds(col, L)][...] = (
                        dis * (agg + hh2) + b2)

            pltpu.sync_copy(rows_a.at[pl.ds(0, rps)],
                            out_hbm.at[pl.ds(row0, rps)])

    return agg_kernel


def _tc_matmul(x_pad, W1, NP, H1):
    def body(x_ref, w_ref, o_ref):
        o_ref[...] = jnp.dot(x_ref[...], w_ref[...],
                             preferred_element_type=jnp.float32)

    return pl.pallas_call(
        body,
        out_shape=jax.ShapeDtypeStruct((NP, H1), jnp.float32),
    )(x_pad, W1)


def _tc_mid(dis16, pa, pb, h1, W2, b1, NP, H2):
    def body(dis16_ref, pa_ref, pb_ref, h1_ref, w2_ref, b1_ref, hh2_ref):
        dis = (jnp.sum(dis16_ref[...], axis=1) * (1.0 / DIS_W))[:, None]
        pa_v = pa_ref[...]
        pb_v = pb_ref[...]
        agg = jnp.concatenate([pa_v[0] + pa_v[1], pb_v[0] + pb_v[1]], axis=1)
        pre = dis * agg + dis * dis * h1_ref[...] + b1_ref[...]
        h = jnp.maximum(pre, 0.0)
        h2 = jnp.dot(h, w2_ref[...], preferred_element_type=jnp.float32)
        hh2_ref[...] = dis * h2

    return pl.pallas_call(
        body,
        out_shape=jax.ShapeDtypeStruct((NP, H2), jnp.float32),
    )(dis16, pa, pb, h1, W2, b1)


def kernel(x, edge_index, W1, b1, W2, b2):
    N, D_IN = x.shape
    E = edge_index.shape[1]
    H1 = W1.shape[1]
    H2 = W2.shape[1]
    assert H1 == 2 * L and H2 % L == 0
    NP = -(-N // 128) * 128

    cpw = -(-E // (NW * CH * 2 * G)) * 2 * G
    E_pad = NW * CH * cpw

    src = edge_index[0]
    dst = edge_index[1]
    pad_idx = jnp.full((E_pad - E,), N, jnp.int32)
    src_p = jnp.concatenate([src, pad_idx]).reshape(NW, cpw, CH)
    dst_p = jnp.concatenate([dst, pad_idx]).reshape(NW, cpw, CH)
    x_pad = jnp.concatenate([x, jnp.zeros((NP - N, D_IN), x.dtype)])

    deg_kernel = _make_deg_kernel(NP, cpw)
    layer1_kernel = _make_layer1_kernel(NP, cpw)
    agg2_kernel = _make_agg2_final_kernel(NP, cpw, H2)

    src_p2 = src_p.reshape(NC, NS, cpw, CH).transpose(1, 0, 2, 3)
    src_p2 = src_p2.reshape(NS, 2 * cpw, CH)
    dst_p2 = dst_p.reshape(NC, NS, cpw, CH).transpose(1, 0, 2, 3)
    dst_p2 = dst_p2.reshape(NS, 2 * cpw, CH)
    b2_pad = jnp.broadcast_to(b2.reshape(1, H2), (8, H2))

    degp = deg_kernel(dst_p)
    h1 = _tc_matmul(x_pad, W1, NP, H1)
    pa, pb, dis16 = layer1_kernel(h1, degp, src_p, dst_p)
    hh2 = _tc_mid(dis16, pa, pb, h1, W2, b1.reshape(1, H1), NP, H2)
    out = agg2_kernel(hh2, dis16, src_p2, dst_p2, b2_pad)
    return out[:N]


# speedup vs baseline: 1.0417x; 1.0417x over previous
import functools

import jax
import jax.numpy as jnp
from jax import lax
from jax.experimental import pallas as pl
from jax.experimental.pallas import tpu as pltpu
from jax.experimental.pallas import tpu_sc as plsc

NC = 2
NS = 16
L = 16
NW = NC * NS
CH = 128
G = 8

DIS_W = 16


def _mesh():
    return plsc.VectorSubcoreMesh(core_axis_name="c", subcore_axis_name="s")


_SC_PARAMS = pltpu.CompilerParams(use_tc_tiling_on_sc=False,
                                  needs_layout_passes=False)


def _fill_rows(buf, rows, Dw, value):
    vec = jnp.full((L,), value, jnp.float32)

    @pl.loop(0, rows)
    def _(r):
        @pl.loop(0, Dw, step=L)
        def _(col):
            buf.at[r, pl.ds(col, L)][...] = vec


def _rsqrt_nr(d):
    i = plsc.bitcast(d, jnp.int32)
    i = jnp.full(d.shape, 0x5F3759DF, jnp.int32) - lax.shift_right_logical(i, 1)
    y = plsc.bitcast(i, jnp.float32)
    half_d = 0.5 * d
    for _ in range(3):
        y = y * (1.5 - half_d * y * y)
    return y


def _agg_pipeline(table, src_v, dst_v, acc, rows_a, rows_b,
                  sem_ga, sem_gb, sem_sa, sem_sb, S):
    def fire_g(sc_idx, rows, sem):
        for g in range(G):
            pltpu.async_copy(table.at[src_v.at[sc_idx * G + g]],
                             rows.at[pl.ds(g * CH, CH)], sem)

    def drain_g(rows, sem):
        for g in range(G):
            pltpu.make_async_copy(table.at[src_v.at[0]],
                                  rows.at[pl.ds(g * CH, CH)], sem).wait()

    def fire_s(sc_idx, rows, sem):
        for g in range(G):
            pltpu.async_copy(rows.at[pl.ds(g * CH, CH)],
                             acc.at[dst_v.at[sc_idx * G + g]], sem, add=True)

    def drain_s(rows, sem):
        for g in range(G):
            pltpu.make_async_copy(rows.at[pl.ds(g * CH, CH)],
                                  acc.at[dst_v.at[0]], sem).wait()

    fire_g(0, rows_a, sem_ga)

    @pl.loop(0, S // 2)
    def _(t):
        s0 = t * 2
        s1 = s0 + 1
        drain_g(rows_a, sem_ga)

        @pl.when(t > 0)
        def _():
            drain_s(rows_b, sem_sb)

        fire_g(s1, rows_b, sem_gb)
        fire_s(s0, rows_a, sem_sa)
        drain_g(rows_b, sem_gb)
        drain_s(rows_a, sem_sa)

        @pl.when(t + 1 < S // 2)
        def _():
            fire_g(s0 + 2, rows_a, sem_ga)

        fire_s(s1, rows_b, sem_sb)

    drain_s(rows_b, sem_sb)


def _make_deg_kernel(NP, cpw):
    rps = NP // NS

    @functools.partial(
        pl.kernel,
        out_type=jax.ShapeDtypeStruct((NC, NP, DIS_W), jnp.float32),
        mesh=_mesh(),
        scratch_types=[
            pltpu.VMEM((cpw, CH), jnp.int32),
            pltpu.VMEM((CH, DIS_W), jnp.float32),
            pltpu.VMEM((rps, DIS_W), jnp.float32),
            pltpu.VMEM_SHARED((NP, DIS_W), jnp.float32),
            pltpu.SemaphoreType.DMA,
        ],
        compiler_params=_SC_PARAMS,
    )
    def deg_kernel(dst_hbm, out_hbm, dst_v, ones_v, zbuf, acc, sem):
        c = lax.axis_index("c")
        s = lax.axis_index("s")
        _fill_rows(ones_v, CH, DIS_W, 1.0)
        _fill_rows(zbuf, rps, DIS_W, 0.0)
        row0 = pl.multiple_of(s * rps, 8)
        pltpu.sync_copy(zbuf, acc.at[pl.ds(row0, rps)])
        plsc.subcore_barrier()

        w = c * NS + s
        pltpu.sync_copy(dst_hbm.at[w], dst_v)

        @pl.loop(0, cpw)
        def _(j):
            pltpu.async_copy(ones_v, acc.at[dst_v.at[j]], sem, add=True)

        @pl.loop(0, cpw)
        def _(j):
            pltpu.make_async_copy(ones_v, acc.at[dst_v.at[0]], sem).wait()

        plsc.subcore_barrier()
        pltpu.sync_copy(acc.at[pl.ds(row0, rps)],
                        out_hbm.at[c].at[pl.ds(row0, rps)])

    return deg_kernel


def _make_layer1_kernel(NP, cpw):
    rps = NP // NS
    S = cpw // G

    @functools.partial(
        pl.kernel,
        out_type=(
            jax.ShapeDtypeStruct((NC, NP, L), jnp.float32),
            jax.ShapeDtypeStruct((NC, NP, L), jnp.float32),
            jax.ShapeDtypeStruct((NP, DIS_W), jnp.float32),
        ),
        mesh=_mesh(),
        scratch_types=[
            pltpu.VMEM((cpw, CH), jnp.int32),
            pltpu.VMEM((cpw, CH), jnp.int32),
            pltpu.VMEM((G * CH, L), jnp.float32),
            pltpu.VMEM((G * CH, L), jnp.float32),
            pltpu.VMEM((NP // NS, 2 * L), jnp.float32),
            pltpu.VMEM((NP // NS, DIS_W), jnp.float32),
            pltpu.VMEM((NP // NS, DIS_W), jnp.float32),
            pltpu.VMEM_SHARED((NP, L), jnp.float32),
            pltpu.VMEM_SHARED((NP, L), jnp.float32),
            pltpu.VMEM_SHARED((NP, L), jnp.float32),
            pltpu.SemaphoreType.DMA,
            pltpu.SemaphoreType.DMA,
            pltpu.SemaphoreType.DMA,
            pltpu.SemaphoreType.DMA,
        ],
        compiler_params=_SC_PARAMS,
    )
    def layer1_kernel(h1_hbm, degp_hbm, src_hbm, dst_hbm,
                      outa_hbm, outb_hbm, dis_hbm,
                      src_v, dst_v, rows_a, rows_b, h_v, dis_v, deg1_v,
                      acc, table_a, table_b,
                      sem_ga, sem_gb, sem_sa, sem_sb):
        c = lax.axis_index("c")
        s = lax.axis_index("s")
        row0 = pl.multiple_of(s * rps, 8)
        acc_slice = acc.at[pl.ds(row0, rps)]

        _fill_rows(rows_a, rps, L, 0.0)
        pltpu.sync_copy(rows_a.at[pl.ds(0, rps)], acc_slice)

        pltpu.sync_copy(degp_hbm.at[0].at[pl.ds(row0, rps)], dis_v)
        pltpu.sync_copy(degp_hbm.at[1].at[pl.ds(row0, rps)], deg1_v)
        pltpu.sync_copy(h1_hbm.at[pl.ds(row0, rps)], h_v)

        @pl.loop(0, rps)
        def _(r):
            d = (dis_v.at[r, pl.ds(0, L)][...]
                 + deg1_v.at[r, pl.ds(0, L)][...] + 1.0)
            dis = _rsqrt_nr(d)
            dis_v.at[r, pl.ds(0, L)][...] = dis
            rows_a.at[r, pl.ds(0, L)][...] = h_v.at[r, pl.ds(0, L)][...] * dis
            rows_b.at[r, pl.ds(0, L)][...] = h_v.at[r, pl.ds(L, L)][...] * dis

        pltpu.sync_copy(rows_a.at[pl.ds(0, rps)], table_a.at[pl.ds(row0, rps)])
        pltpu.sync_copy(rows_b.at[pl.ds(0, rps)], table_b.at[pl.ds(row0, rps)])

        @pl.when(c == s % 2)
        def _():
            pltpu.sync_copy(dis_v, dis_hbm.at[pl.ds(row0, rps)])

        w = c * NS + s
        pltpu.sync_copy(src_hbm.at[w], src_v)
        pltpu.sync_copy(dst_hbm.at[w], dst_v)
        plsc.subcore_barrier()

        _agg_pipeline(table_a, src_v, dst_v, acc, rows_a, rows_b,
                      sem_ga, sem_gb, sem_sa, sem_sb, S)

        plsc.subcore_barrier()
        pltpu.sync_copy(acc_slice, outa_hbm.at[c].at[pl.ds(row0, rps)])

        _fill_rows(rows_a, rps, L, 0.0)
        pltpu.sync_copy(rows_a.at[pl.ds(0, rps)], acc_slice)
        plsc.subcore_barrier()

        _agg_pipeline(table_b, src_v, dst_v, acc, rows_a, rows_b,
                      sem_ga, sem_gb, sem_sa, sem_sb, S)

        plsc.subcore_barrier()
        pltpu.sync_copy(acc_slice, outb_hbm.at[c].at[pl.ds(row0, rps)])

    return layer1_kernel


def _make_agg_kernel(NP, cpw, Dw):
    rps = NP // NS
    S = cpw // G

    @functools.partial(
        pl.kernel,
        out_type=jax.ShapeDtypeStruct((NC, NP, Dw), jnp.float32),
        mesh=_mesh(),
        scratch_types=[
            pltpu.VMEM((cpw, CH), jnp.int32),
            pltpu.VMEM((cpw, CH), jnp.int32),
            pltpu.VMEM((G * CH, Dw), jnp.float32),
            pltpu.VMEM((G * CH, Dw), jnp.float32),
            pltpu.VMEM_SHARED((NP, Dw), jnp.float32),
            pltpu.VMEM_SHARED((NP, Dw), jnp.float32),
            pltpu.SemaphoreType.DMA,
            pltpu.SemaphoreType.DMA,
            pltpu.SemaphoreType.DMA,
            pltpu.SemaphoreType.DMA,
        ],
        compiler_params=_SC_PARAMS,
    )
    def agg_kernel(table_hbm, src_hbm, dst_hbm, out_hbm,
                   src_v, dst_v, rows_a, rows_b, acc, table,
                   sem_ga, sem_gb, sem_sa, sem_sb):
        c = lax.axis_index("c")
        s = lax.axis_index("s")
        row0 = pl.multiple_of(s * rps, 8)

        _fill_rows(rows_a, rps, Dw, 0.0)
        pltpu.sync_copy(rows_a.at[pl.ds(0, rps)], acc.at[pl.ds(row0, rps)])
        pltpu.sync_copy(table_hbm.at[pl.ds(row0, rps)],
                        table.at[pl.ds(row0, rps)])

        w = c * NS + s
        pltpu.sync_copy(src_hbm.at[w], src_v)
        pltpu.sync_copy(dst_hbm.at[w], dst_v)
        plsc.subcore_barrier()

        _agg_pipeline(table, src_v, dst_v, acc, rows_a, rows_b,
                      sem_ga, sem_gb, sem_sa, sem_sb, S)

        plsc.subcore_barrier()
        pltpu.sync_copy(acc.at[pl.ds(row0, rps)],
                        out_hbm.at[c].at[pl.ds(row0, rps)])

    return agg_kernel


def _tc_matmul(x_pad, W1, NP, H1):
    def body(x_ref, w_ref, o_ref):
        o_ref[...] = jnp.dot(x_ref[...], w_ref[...],
                             preferred_element_type=jnp.float32)

    return pl.pallas_call(
        body,
        out_shape=jax.ShapeDtypeStruct((NP, H1), jnp.float32),
    )(x_pad, W1)


def _tc_mid(dis16, pa, pb, h1, W2, b1, NP, H2):
    def body(dis16_ref, pa_ref, pb_ref, h1_ref, w2_ref, b1_ref,
             dis_ref, hh2_ref):
        dis = (jnp.sum(dis16_ref[...], axis=1) * (1.0 / DIS_W))[:, None]
        dis_ref[...] = dis
        pa_v = pa_ref[...]
        pb_v = pb_ref[...]
        agg = jnp.concatenate([pa_v[0] + pa_v[1], pb_v[0] + pb_v[1]], axis=1)
        pre = dis * agg + dis * dis * h1_ref[...] + b1_ref[...]
        h = jnp.maximum(pre, 0.0)
        h2 = jnp.dot(h, w2_ref[...], preferred_element_type=jnp.float32)
        hh2_ref[...] = dis * h2

    return pl.pallas_call(
        body,
        out_shape=(
            jax.ShapeDtypeStruct((NP, 1), jnp.float32),
            jax.ShapeDtypeStruct((NP, H2), jnp.float32),
        ),
    )(dis16, pa, pb, h1, W2, b1)


def _tc_out(p2, hh2, dis, b2, NP, H2):
    def body(p_ref, hh2_ref, dis_ref, b2_ref, o_ref):
        p = p_ref[...]
        o_ref[...] = dis_ref[...] * (p[0] + p[1] + hh2_ref[...]) + b2_ref[...]

    return pl.pallas_call(
        body,
        out_shape=jax.ShapeDtypeStruct((NP, H2), jnp.float32),
    )(p2, hh2, dis, b2)


def kernel(x, edge_index, W1, b1, W2, b2):
    N, D_IN = x.shape
    E = edge_index.shape[1]
    H1 = W1.shape[1]
    H2 = W2.shape[1]
    assert H1 == 2 * L and H2 % L == 0
    NP = -(-N // 128) * 128

    cpw = -(-E // (NW * CH * 2 * G)) * 2 * G
    E_pad = NW * CH * cpw

    src = edge_index[0]
    dst = edge_index[1]
    pad_idx = jnp.full((E_pad - E,), N, jnp.int32)
    src_p = jnp.concatenate([src, pad_idx]).reshape(NW, cpw, CH)
    dst_p = jnp.concatenate([dst, pad_idx]).reshape(NW, cpw, CH)
    x_pad = jnp.concatenate([x, jnp.zeros((NP - N, D_IN), x.dtype)])

    deg_kernel = _make_deg_kernel(NP, cpw)
    layer1_kernel = _make_layer1_kernel(NP, cpw)
    agg2_kernel = _make_agg_kernel(NP, cpw, H2)

    degp = deg_kernel(dst_p)
    h1 = _tc_matmul(x_pad, W1, NP, H1)
    pa, pb, dis16 = layer1_kernel(h1, degp, src_p, dst_p)
    dis, hh2 = _tc_mid(dis16, pa, pb, h1, W2, b1.reshape(1, H1), NP, H2)
    p2 = agg2_kernel(hh2, src_p, dst_p)
    out = _tc_out(p2, hh2, dis, b2.reshape(1, H2), NP, H2)
    return out[:N]


# speedup vs baseline: 1.0478x; 1.0059x over previous
import functools

import jax
import jax.numpy as jnp
from jax import lax
from jax.experimental import pallas as pl
from jax.experimental.pallas import tpu as pltpu
from jax.experimental.pallas import tpu_sc as plsc

NC = 2
NS = 16
L = 16
NW = NC * NS
CH = 128
G = 8

DIS_W = 16


def _mesh():
    return plsc.VectorSubcoreMesh(core_axis_name="c", subcore_axis_name="s")


_SC_PARAMS = pltpu.CompilerParams(use_tc_tiling_on_sc=False,
                                  needs_layout_passes=False)


def _fill_rows(buf, rows, Dw, value):
    vec = jnp.full((L,), value, jnp.float32)

    @pl.loop(0, rows)
    def _(r):
        @pl.loop(0, Dw, step=L)
        def _(col):
            buf.at[r, pl.ds(col, L)][...] = vec


def _rsqrt_nr(d):
    i = plsc.bitcast(d, jnp.int32)
    i = jnp.full(d.shape, 0x5F3759DF, jnp.int32) - lax.shift_right_logical(i, 1)
    y = plsc.bitcast(i, jnp.float32)
    half_d = 0.5 * d
    for _ in range(3):
        y = y * (1.5 - half_d * y * y)
    return y


def _agg_pipeline(table, src_v, dst_v, acc, rows_a, rows_b,
                  sem_ga, sem_gb, sem_sa, sem_sb, S):
    def fire_g(sc_idx, rows, sem):
        for g in range(G):
            pltpu.async_copy(table.at[src_v.at[sc_idx * G + g]],
                             rows.at[pl.ds(g * CH, CH)], sem)

    def drain_g(rows, sem):
        for g in range(G):
            pltpu.make_async_copy(table.at[src_v.at[0]],
                                  rows.at[pl.ds(g * CH, CH)], sem).wait()

    def fire_s(sc_idx, rows, sem):
        for g in range(G):
            pltpu.async_copy(rows.at[pl.ds(g * CH, CH)],
                             acc.at[dst_v.at[sc_idx * G + g]], sem, add=True)

    def drain_s(rows, sem):
        for g in range(G):
            pltpu.make_async_copy(rows.at[pl.ds(g * CH, CH)],
                                  acc.at[dst_v.at[0]], sem).wait()

    fire_g(0, rows_a, sem_ga)

    @pl.loop(0, S // 2)
    def _(t):
        s0 = t * 2
        s1 = s0 + 1
        drain_g(rows_a, sem_ga)

        @pl.when(t > 0)
        def _():
            drain_s(rows_b, sem_sb)

        fire_g(s1, rows_b, sem_gb)
        fire_s(s0, rows_a, sem_sa)
        drain_g(rows_b, sem_gb)
        drain_s(rows_a, sem_sa)

        @pl.when(t + 1 < S // 2)
        def _():
            fire_g(s0 + 2, rows_a, sem_ga)

        fire_s(s1, rows_b, sem_sb)

    drain_s(rows_b, sem_sb)


def _make_deg_kernel(NP, cpw):
    rps = NP // NS

    @functools.partial(
        pl.kernel,
        out_type=jax.ShapeDtypeStruct((NC, NP, DIS_W), jnp.float32),
        mesh=_mesh(),
        scratch_types=[
            pltpu.VMEM((cpw, CH), jnp.int32),
            pltpu.VMEM((CH, DIS_W), jnp.float32),
            pltpu.VMEM((rps, DIS_W), jnp.float32),
            pltpu.VMEM_SHARED((NP, DIS_W), jnp.float32),
            pltpu.SemaphoreType.DMA,
        ],
        compiler_params=_SC_PARAMS,
    )
    def deg_kernel(dst_hbm, out_hbm, dst_v, ones_v, zbuf, acc, sem):
        c = lax.axis_index("c")
        s = lax.axis_index("s")
        _fill_rows(ones_v, CH, DIS_W, 1.0)
        _fill_rows(zbuf, rps, DIS_W, 0.0)
        row0 = pl.multiple_of(s * rps, 8)
        pltpu.sync_copy(zbuf, acc.at[pl.ds(row0, rps)])
        plsc.subcore_barrier()

        w = c * NS + s
        pltpu.sync_copy(dst_hbm.at[w], dst_v)

        @pl.loop(0, cpw)
        def _(j):
            pltpu.async_copy(ones_v, acc.at[dst_v.at[j]], sem, add=True)

        @pl.loop(0, cpw)
        def _(j):
            pltpu.make_async_copy(ones_v, acc.at[dst_v.at[0]], sem).wait()

        plsc.subcore_barrier()
        pltpu.sync_copy(acc.at[pl.ds(row0, rps)],
                        out_hbm.at[c].at[pl.ds(row0, rps)])

    return deg_kernel


def _make_layer1_kernel(NP, cpw):
    rps = NP // NS
    S = cpw // G

    @functools.partial(
        pl.kernel,
        out_type=(
            jax.ShapeDtypeStruct((NC, NP, L), jnp.float32),
            jax.ShapeDtypeStruct((NC, NP, L), jnp.float32),
            jax.ShapeDtypeStruct((NP, DIS_W), jnp.float32),
        ),
        mesh=_mesh(),
        scratch_types=[
            pltpu.VMEM((cpw, CH), jnp.int32),
            pltpu.VMEM((cpw, CH), jnp.int32),
            pltpu.VMEM((G * CH, L), jnp.float32),
            pltpu.VMEM((G * CH, L), jnp.float32),
            pltpu.VMEM((NP // NS, 2 * L), jnp.float32),
            pltpu.VMEM((NP // NS, DIS_W), jnp.float32),
            pltpu.VMEM((NP // NS, DIS_W), jnp.float32),
            pltpu.VMEM_SHARED((NP, L), jnp.float32),
            pltpu.VMEM_SHARED((NP, L), jnp.float32),
            pltpu.VMEM_SHARED((NP, L), jnp.float32),
            pltpu.SemaphoreType.DMA,
            pltpu.SemaphoreType.DMA,
            pltpu.SemaphoreType.DMA,
            pltpu.SemaphoreType.DMA,
        ],
        compiler_params=_SC_PARAMS,
    )
    def layer1_kernel(h1_hbm, degp_hbm, src_hbm, dst_hbm,
                      outa_hbm, outb_hbm, dis_hbm,
                      src_v, dst_v, rows_a, rows_b, h_v, dis_v, deg1_v,
                      acc, table_a, table_b,
                      sem_ga, sem_gb, sem_sa, sem_sb):
        c = lax.axis_index("c")
        s = lax.axis_index("s")
        row0 = pl.multiple_of(s * rps, 8)
        acc_slice = acc.at[pl.ds(row0, rps)]

        _fill_rows(rows_a, rps, L, 0.0)
        pltpu.sync_copy(rows_a.at[pl.ds(0, rps)], acc_slice)

        pltpu.sync_copy(degp_hbm.at[0].at[pl.ds(row0, rps)], dis_v)
        pltpu.sync_copy(degp_hbm.at[1].at[pl.ds(row0, rps)], deg1_v)
        pltpu.sync_copy(h1_hbm.at[pl.ds(row0, rps)], h_v)

        @pl.loop(0, rps)
        def _(r):
            d = (dis_v.at[r, pl.ds(0, L)][...]
                 + deg1_v.at[r, pl.ds(0, L)][...] + 1.0)
            dis = _rsqrt_nr(d)
            dis_v.at[r, pl.ds(0, L)][...] = dis
            rows_a.at[r, pl.ds(0, L)][...] = h_v.at[r, pl.ds(0, L)][...] * dis
            rows_b.at[r, pl.ds(0, L)][...] = h_v.at[r, pl.ds(L, L)][...] * dis

        pltpu.sync_copy(rows_a.at[pl.ds(0, rps)], table_a.at[pl.ds(row0, rps)])
        pltpu.sync_copy(rows_b.at[pl.ds(0, rps)], table_b.at[pl.ds(row0, rps)])

        @pl.when(c == s % 2)
        def _():
            pltpu.sync_copy(dis_v, dis_hbm.at[pl.ds(row0, rps)])

        w = c * NS + s
        pltpu.sync_copy(src_hbm.at[w], src_v)
        pltpu.sync_copy(dst_hbm.at[w], dst_v)
        plsc.subcore_barrier()

        _agg_pipeline(table_a, src_v, dst_v, acc, rows_a, rows_b,
                      sem_ga, sem_gb, sem_sa, sem_sb, S)

        plsc.subcore_barrier()
        pltpu.sync_copy(acc_slice, outa_hbm.at[c].at[pl.ds(row0, rps)])

        _fill_rows(rows_a, rps, L, 0.0)
        pltpu.sync_copy(rows_a.at[pl.ds(0, rps)], acc_slice)
        plsc.subcore_barrier()

        _agg_pipeline(table_b, src_v, dst_v, acc, rows_a, rows_b,
                      sem_ga, sem_gb, sem_sa, sem_sb, S)

        plsc.subcore_barrier()
        pltpu.sync_copy(acc_slice, outb_hbm.at[c].at[pl.ds(row0, rps)])

    return layer1_kernel


def _make_agg_kernel(NP, cpw, Dw):
    rps = NP // NS
    S = cpw // G

    @functools.partial(
        pl.kernel,
        out_type=jax.ShapeDtypeStruct((NC, NP, Dw), jnp.float32),
        mesh=_mesh(),
        scratch_types=[
            pltpu.VMEM((cpw, CH), jnp.int32),
            pltpu.VMEM((cpw, CH), jnp.int32),
            pltpu.VMEM((G * CH, Dw), jnp.float32),
            pltpu.VMEM((G * CH, Dw), jnp.float32),
            pltpu.VMEM_SHARED((NP, Dw), jnp.float32),
            pltpu.VMEM_SHARED((NP, Dw), jnp.float32),
            pltpu.SemaphoreType.DMA,
            pltpu.SemaphoreType.DMA,
            pltpu.SemaphoreType.DMA,
            pltpu.SemaphoreType.DMA,
        ],
        compiler_params=_SC_PARAMS,
    )
    def agg_kernel(table_hbm, src_hbm, dst_hbm, out_hbm,
                   src_v, dst_v, rows_a, rows_b, acc, table,
                   sem_ga, sem_gb, sem_sa, sem_sb):
        c = lax.axis_index("c")
        s = lax.axis_index("s")
        row0 = pl.multiple_of(s * rps, 8)

        _fill_rows(rows_a, rps, Dw, 0.0)
        pltpu.sync_copy(rows_a.at[pl.ds(0, rps)], acc.at[pl.ds(row0, rps)])
        pltpu.sync_copy(table_hbm.at[pl.ds(row0, rps)],
                        table.at[pl.ds(row0, rps)])

        w = c * NS + s
        pltpu.sync_copy(src_hbm.at[w], src_v)
        pltpu.sync_copy(dst_hbm.at[w], dst_v)
        plsc.subcore_barrier()

        _agg_pipeline(table, src_v, dst_v, acc, rows_a, rows_b,
                      sem_ga, sem_gb, sem_sa, sem_sb, S)

        plsc.subcore_barrier()
        pltpu.sync_copy(acc.at[pl.ds(row0, rps)],
                        out_hbm.at[c].at[pl.ds(row0, rps)])

    return agg_kernel


def _tc_matmul(x, W1, N, NP, H1):
    def body(x_ref, w_ref, o_ref):
        h = jnp.dot(x_ref[...], w_ref[...],
                    preferred_element_type=jnp.float32)
        o_ref[...] = jnp.concatenate(
            [h, jnp.zeros((NP - N, H1), jnp.float32)])

    return pl.pallas_call(
        body,
        out_shape=jax.ShapeDtypeStruct((NP, H1), jnp.float32),
    )(x, W1)


def _tc_mid(dis16, pa, pb, h1, W2, b1, NP, H2):
    def body(dis16_ref, pa_ref, pb_ref, h1_ref, w2_ref, b1_ref,
             dis_ref, hh2_ref):
        dis = (jnp.sum(dis16_ref[...], axis=1) * (1.0 / DIS_W))[:, None]
        dis_ref[...] = dis
        pa_v = pa_ref[...]
        pb_v = pb_ref[...]
        agg = jnp.concatenate([pa_v[0] + pa_v[1], pb_v[0] + pb_v[1]], axis=1)
        pre = dis * agg + dis * dis * h1_ref[...] + b1_ref[...]
        h = jnp.maximum(pre, 0.0)
        h2 = jnp.dot(h, w2_ref[...], preferred_element_type=jnp.float32)
        hh2_ref[...] = dis * h2

    return pl.pallas_call(
        body,
        out_shape=(
            jax.ShapeDtypeStruct((NP, 1), jnp.float32),
            jax.ShapeDtypeStruct((NP, H2), jnp.float32),
        ),
    )(dis16, pa, pb, h1, W2, b1)


def _tc_out(p2, hh2, dis, b2, N, H2):
    def body(p_ref, hh2_ref, dis_ref, b2_ref, o_ref):
        p = p_ref[...]
        out = dis_ref[...] * (p[0] + p[1] + hh2_ref[...]) + b2_ref[...]
        o_ref[...] = out[:N]

    return pl.pallas_call(
        body,
        out_shape=jax.ShapeDtypeStruct((N, H2), jnp.float32),
    )(p2, hh2, dis, b2)


def kernel(x, edge_index, W1, b1, W2, b2):
    N, D_IN = x.shape
    E = edge_index.shape[1]
    H1 = W1.shape[1]
    H2 = W2.shape[1]
    assert H1 == 2 * L and H2 % L == 0
    NP = -(-N // 128) * 128

    cpw = -(-E // (NW * CH * 2 * G)) * 2 * G
    E_pad = NW * CH * cpw

    src = edge_index[0]
    dst = edge_index[1]
    pad_idx = jnp.full((E_pad - E,), N, jnp.int32)
    src_p = jnp.concatenate([src, pad_idx]).reshape(NW, cpw, CH)
    dst_p = jnp.concatenate([dst, pad_idx]).reshape(NW, cpw, CH)

    deg_kernel = _make_deg_kernel(NP, cpw)
    layer1_kernel = _make_layer1_kernel(NP, cpw)
    agg2_kernel = _make_agg_kernel(NP, cpw, H2)

    degp = deg_kernel(dst_p)
    h1 = _tc_matmul(x, W1, N, NP, H1)
    pa, pb, dis16 = layer1_kernel(h1, degp, src_p, dst_p)
    dis, hh2 = _tc_mid(dis16, pa, pb, h1, W2, b1.reshape(1, H1), NP, H2)
    p2 = agg2_kernel(hh2, src_p, dst_p)
    return _tc_out(p2, hh2, dis, b2.reshape(1, H2), N, H2)
